# baseline (speedup 1.0000x reference)
import functools

import jax
import jax.numpy as jnp
from jax import lax
from jax.experimental import pallas as pl
from jax.experimental.pallas import tpu as pltpu
from jax.experimental.pallas import tpu_sc as plsc

F = 128
LANES = 16
NC = 2
NS = 16
NW = NC * NS
CHUNK = 128
NB_DEG = 8
NB_SP = 2


@functools.lru_cache(maxsize=None)
def _make_deg(EP, N, NPAD):
    chunks = EP // NS // CHUNK
    groups = chunks // NB_DEG
    mesh = plsc.VectorSubcoreMesh(core_axis_name="c", subcore_axis_name="s")

    @functools.partial(
        pl.kernel,
        mesh=mesh,
        out_type=jax.ShapeDtypeStruct((2, NPAD), jnp.float32),
        scratch_types=[
            pltpu.VMEM((chunks, CHUNK), jnp.int32),
            pltpu.VMEM((CHUNK,), jnp.float32),
            pltpu.VMEM_SHARED((NPAD,), jnp.float32),
            pltpu.SemaphoreType.DMA((NB_DEG,)),
        ],
    )
    def deg(ei_hbm, zeros_hbm, out_hbm, idx_all, ones_v, acc, sems):
        c = lax.axis_index("c")
        s = lax.axis_index("s")
        for i in range(CHUNK // LANES):
            ones_v[pl.ds(i * LANES, LANES)] = jnp.full(
                (LANES,), 1.0, jnp.float32
            )

        @pl.when(s == 0)
        def _():
            pltpu.sync_copy(zeros_hbm, acc)

        pltpu.sync_copy(
            ei_hbm.at[c, pl.ds(pl.multiple_of(s * chunks, 8), chunks)],
            idx_all,
        )
        plsc.subcore_barrier()

        def body(j, carry):
            ds = []
            for b in range(NB_DEG):
                row = j * NB_DEG + b
                ds.append(pltpu.async_copy(
                    ones_v, acc.at[idx_all.at[row]], sems.at[b], add=True
                ))
            for d in ds:
                d.wait()
            return carry

        lax.fori_loop(0, groups, body, 0)
        plsc.subcore_barrier()

        @pl.when(s == 0)
        def _():
            pltpu.sync_copy(acc, out_hbm.at[c])

    return deg


@functools.lru_cache(maxsize=None)
def _make_spmm(EP, N, NPAD):
    chunks = EP // NW // CHUNK
    groups = chunks // NB_SP
    zrows = NPAD // 8
    orows = 1000
    mesh = plsc.VectorSubcoreMesh(core_axis_name="c", subcore_axis_name="s")

    @functools.partial(
        pl.kernel,
        mesh=mesh,
        out_type=jax.ShapeDtypeStruct((2, N, F), jnp.float32),
        scratch_types=[
            pltpu.VMEM((CHUNK,), jnp.int32),
            pltpu.VMEM((CHUNK,), jnp.int32),
            pltpu.VMEM((CHUNK, F), jnp.float32),
            pltpu.VMEM_SHARED((NPAD, F), jnp.float32),
            pltpu.SemaphoreType.DMA,
        ],
    )
    def spmm(h_hbm, src_hbm, dst_hbm, zeros_hbm, out_hbm,
             sidx, didx, rows_v, acc, gsem):
        c = lax.axis_index("c")
        s = lax.axis_index("s")
        wid = c * NS + s
        base = wid * chunks

        @pl.when(s < 8)
        def _():
            pltpu.sync_copy(
                zeros_hbm, acc.at[pl.ds(pl.multiple_of(s * zrows, 8), zrows)]
            )

        plsc.subcore_barrier()

        def body(g, carry):
            off = pl.multiple_of((base + g) * CHUNK, 8)
            pltpu.sync_copy(src_hbm.at[pl.ds(off, CHUNK)], sidx)
            pltpu.sync_copy(dst_hbm.at[pl.ds(off, CHUNK)], didx)
            pltpu.async_copy(h_hbm.at[sidx], rows_v, gsem).wait()
            pltpu.sync_copy(rows_v, acc.at[didx], add=True)
            return carry

        lax.fori_loop(0, chunks, body, 0)
        plsc.subcore_barrier()

        @pl.when(s < N // orows)
        def _():
            obase = pl.multiple_of(s * orows, 8)
            pltpu.sync_copy(
                acc.at[pl.ds(obase, orows)],
                out_hbm.at[c, pl.ds(obase, orows)],
            )

    return spmm


def _tc1_body(x_ref, d_ref, w_ref, o_ref):
    s = lax.rsqrt(jnp.maximum(d_ref[...], 1.0))
    o_ref[...] = jnp.dot(
        x_ref[...] * s, w_ref[...], preferred_element_type=jnp.float32
    )


def _tc2_body(p0_ref, p1_ref, din_ref, dout_ref, b_ref, w_ref, o_ref):
    t = (p0_ref[...] + p1_ref[...]) * lax.rsqrt(
        jnp.maximum(din_ref[...], 1.0)
    ) + b_ref[...]
    t = jnp.maximum(t, 0.0)
    t = t * lax.rsqrt(jnp.maximum(dout_ref[...], 1.0))
    o_ref[...] = jnp.dot(t, w_ref[...], preferred_element_type=jnp.float32)


def _tc3_body(q0_ref, q1_ref, din_ref, b_ref, o_ref):
    o_ref[...] = (q0_ref[...] + q1_ref[...]) * lax.rsqrt(
        jnp.maximum(din_ref[...], 1.0)
    ) + b_ref[...]


def _row_spec(R):
    return pl.BlockSpec((R, F), lambda i: (i, 0))


def _deg_spec(R):
    return pl.BlockSpec((R, 1), lambda i: (i, 0))


def _full_spec(shape):
    return pl.BlockSpec(shape, lambda i: (0,) * len(shape))


def _tc1(x, dout, W, R):
    n = x.shape[0]
    return pl.pallas_call(
        _tc1_body,
        grid=(n // R,),
        in_specs=[_row_spec(R), _deg_spec(R), _full_spec((F, F))],
        out_specs=_row_spec(R),
        out_shape=jax.ShapeDtypeStruct((n, F), jnp.float32),
    )(x, dout, W)


def _tc2(p0, p1, din, dout, b, W, R):
    n = p0.shape[0]
    return pl.pallas_call(
        _tc2_body,
        grid=(n // R,),
        in_specs=[_row_spec(R), _row_spec(R), _deg_spec(R), _deg_spec(R),
                  _full_spec((1, F)), _full_spec((F, F))],
        out_specs=_row_spec(R),
        out_shape=jax.ShapeDtypeStruct((n, F), jnp.float32),
    )(p0, p1, din, dout, b, W)


def _tc3(q0, q1, din, b, R):
    n = q0.shape[0]
    return pl.pallas_call(
        _tc3_body,
        grid=(n // R,),
        in_specs=[_row_spec(R), _row_spec(R), _deg_spec(R),
                  _full_spec((1, F))],
        out_specs=_row_spec(R),
        out_shape=jax.ShapeDtypeStruct((n, F), jnp.float32),
    )(q0, q1, din, b)


def kernel(in_feat, edge_index, W1, b1, W2, b2):
    N = in_feat.shape[0]
    E = edge_index.shape[1]

    unit = NS * CHUNK * 8
    EP = -(-E // unit) * unit
    NPAD = 128 * (-(-(N + 1) // 128))
    pad = EP - E

    src = edge_index[0]
    dst = edge_index[1]
    if pad:
        src_g = jnp.concatenate([src, jnp.zeros((pad,), jnp.int32)])
        dst_p = jnp.concatenate([dst, jnp.full((pad,), N, jnp.int32)])
        src_d = jnp.concatenate([src, jnp.full((pad,), N, jnp.int32)])
    else:
        src_g, dst_p, src_d = src, dst, src
    rows = EP // CHUNK
    ei_deg = jnp.stack(
        [src_d.reshape(rows, CHUNK), dst_p.reshape(rows, CHUNK)]
    )

    zeros_deg = jnp.zeros((NPAD,), jnp.float32)
    zeros_rows = jnp.zeros((NPAD // 8, F), jnp.float32)

    degs = _make_deg(EP, N, NPAD)(ei_deg, zeros_deg)
    dout = degs[0, :N, None]
    din = degs[1, :N, None]

    R = 1000 if N % 1000 == 0 else N
    b1r = b1[None, :]
    b2r = b2[None, :]

    spmm = _make_spmm(EP, N, NPAD)

    h1 = _tc1(in_feat, dout, W1, R)
    P = spmm(h1, src_g, dst_p, zeros_rows)
    h2 = _tc2(P[0], P[1], din, dout, b1r, W2, R)
    Q = spmm(h2, src_g, dst_p, zeros_rows)
    return _tc3(Q[0], Q[1], din, b2r, R)


# speedup vs baseline: 2.9704x; 2.9704x over previous
import functools

import jax
import jax.numpy as jnp
from jax import lax
from jax.experimental import pallas as pl
from jax.experimental.pallas import tpu as pltpu
from jax.experimental.pallas import tpu_sc as plsc

F = 128
LANES = 16
NC = 2
NS = 16
NW = NC * NS
CHUNK = 128
NB_DEG = 8
NB_SP = 2


@functools.lru_cache(maxsize=None)
def _make_deg(EP, N, NPAD):
    chunks = EP // NS // CHUNK
    groups = chunks // NB_DEG
    mesh = plsc.VectorSubcoreMesh(core_axis_name="c", subcore_axis_name="s")

    @functools.partial(
        pl.kernel,
        mesh=mesh,
        out_type=jax.ShapeDtypeStruct((2, NPAD), jnp.float32),
        scratch_types=[
            pltpu.VMEM((chunks, CHUNK), jnp.int32),
            pltpu.VMEM((CHUNK,), jnp.float32),
            pltpu.VMEM_SHARED((NPAD,), jnp.float32),
            pltpu.SemaphoreType.DMA((NB_DEG,)),
        ],
    )
    def deg(ei_hbm, zeros_hbm, out_hbm, idx_all, ones_v, acc, sems):
        c = lax.axis_index("c")
        s = lax.axis_index("s")
        for i in range(CHUNK // LANES):
            ones_v[pl.ds(i * LANES, LANES)] = jnp.full(
                (LANES,), 1.0, jnp.float32
            )

        @pl.when(s == 0)
        def _():
            pltpu.sync_copy(zeros_hbm, acc)

        pltpu.sync_copy(
            ei_hbm.at[c, pl.ds(pl.multiple_of(s * chunks, 8), chunks)],
            idx_all,
        )
        plsc.subcore_barrier()

        def body(j, carry):
            ds = []
            for b in range(NB_DEG):
                row = j * NB_DEG + b
                ds.append(pltpu.async_copy(
                    ones_v, acc.at[idx_all.at[row]], sems.at[b], add=True
                ))
            for d in ds:
                d.wait()
            return carry

        lax.fori_loop(0, groups, body, 0)
        plsc.subcore_barrier()

        @pl.when(s == 0)
        def _():
            pltpu.sync_copy(acc, out_hbm.at[c])

    return deg


@functools.lru_cache(maxsize=None)
def _make_spmm(ES, N, NPAD):
    total_quads = ES // (4 * CHUNK)
    q_lo = total_quads // NW
    q_hi = total_quads - q_lo * NW
    zrows = NPAD // 8
    orows = 1000
    mesh = plsc.VectorSubcoreMesh(core_axis_name="c", subcore_axis_name="s")

    @functools.partial(
        pl.kernel,
        mesh=mesh,
        out_type=jax.ShapeDtypeStruct((2, N, F), jnp.float32),
        scratch_types=[
            pltpu.VMEM((2, NB_SP * CHUNK), jnp.int32),
            pltpu.VMEM((2 * NB_SP, CHUNK), jnp.int32),
            pltpu.VMEM((NB_SP, CHUNK, F), jnp.float32),
            pltpu.VMEM_SHARED((NPAD, F), jnp.float32),
            pltpu.SemaphoreType.DMA((2,)),
            pltpu.SemaphoreType.DMA((NB_SP,)),
            pltpu.SemaphoreType.DMA((NB_SP,)),
        ],
    )
    def spmm(h_hbm, src_hbm, dst_hbm, zeros_hbm, out_hbm,
             sidx, didx, rows_v, acc, isems, gsems, ssems):
        c = lax.axis_index("c")
        s = lax.axis_index("s")
        wid = c * NS + s
        nquads = q_lo + jnp.where(wid < q_hi, 1, 0)
        base = (q_lo * wid + jnp.minimum(wid, q_hi)) * 4
        groups = 2 * nquads

        def _idx_descs(par, g):
            off = (base + g * NB_SP) * CHUNK
            off = pl.multiple_of(off, 8)
            ds = [pltpu.make_async_copy(
                src_hbm.at[pl.ds(off, NB_SP * CHUNK)], sidx.at[par],
                isems.at[par],
            )]
            for b in range(NB_SP):
                offb = pl.multiple_of(off + b * CHUNK, 8)
                ds.append(pltpu.make_async_copy(
                    dst_hbm.at[pl.ds(offb, CHUNK)], didx.at[par * NB_SP + b],
                    isems.at[par],
                ))
            return ds

        def fire_idx(par, g):
            for d in _idx_descs(par, g):
                d.start()

        def drain_idx(par, g):
            for d in _idx_descs(par, g):
                d.wait()

        @pl.when(s < 8)
        def _():
            pltpu.sync_copy(
                zeros_hbm, acc.at[pl.ds(pl.multiple_of(s * zrows, 8), zrows)]
            )

        fire_idx(0, 0)
        fire_idx(1, 1)
        plsc.subcore_barrier()

        def phase(par, g):
            drain_idx(par, g)
            gs = []
            for b in range(NB_SP):
                gs.append(pltpu.async_copy(
                    h_hbm.at[sidx.at[par, pl.ds(b * CHUNK, CHUNK)]],
                    rows_v.at[b], gsems.at[b],
                ))
            ss = []
            for b in range(NB_SP):
                gs[b].wait()
                ss.append(pltpu.async_copy(
                    rows_v.at[b], acc.at[didx.at[par * NB_SP + b]],
                    ssems.at[b], add=True,
                ))
            for d in ss:
                d.wait()

            @pl.when(g + 2 < groups)
            def _():
                fire_idx(par, g + 2)

        def body(i, carry):
            phase(0, 2 * i)
            phase(1, 2 * i + 1)
            return carry

        lax.fori_loop(0, nquads, body, 0)
        plsc.subcore_barrier()

        @pl.when(s < N // orows)
        def _():
            obase = pl.multiple_of(s * orows, 8)
            pltpu.sync_copy(
                acc.at[pl.ds(obase, orows)],
                out_hbm.at[c, pl.ds(obase, orows)],
            )

    return spmm


def _tc1_body(x_ref, d_ref, w_ref, o_ref):
    s = lax.rsqrt(jnp.maximum(d_ref[...], 1.0))
    o_ref[...] = jnp.dot(
        x_ref[...] * s, w_ref[...], preferred_element_type=jnp.float32
    )


def _tc2_body(p0_ref, p1_ref, din_ref, dout_ref, b_ref, w_ref, o_ref):
    t = (p0_ref[...] + p1_ref[...]) * lax.rsqrt(
        jnp.maximum(din_ref[...], 1.0)
    ) + b_ref[...]
    t = jnp.maximum(t, 0.0)
    t = t * lax.rsqrt(jnp.maximum(dout_ref[...], 1.0))
    o_ref[...] = jnp.dot(t, w_ref[...], preferred_element_type=jnp.float32)


def _tc3_body(q0_ref, q1_ref, din_ref, b_ref, o_ref):
    o_ref[...] = (q0_ref[...] + q1_ref[...]) * lax.rsqrt(
        jnp.maximum(din_ref[...], 1.0)
    ) + b_ref[...]


def _row_spec(R):
    return pl.BlockSpec((R, F), lambda i: (i, 0))


def _deg_spec(R):
    return pl.BlockSpec((R, 1), lambda i: (i, 0))


def _full_spec(shape):
    return pl.BlockSpec(shape, lambda i: (0,) * len(shape))


def _tc1(x, dout, W, R):
    n = x.shape[0]
    return pl.pallas_call(
        _tc1_body,
        grid=(n // R,),
        in_specs=[_row_spec(R), _deg_spec(R), _full_spec((F, F))],
        out_specs=_row_spec(R),
        out_shape=jax.ShapeDtypeStruct((n, F), jnp.float32),
    )(x, dout, W)


def _tc2(p0, p1, din, dout, b, W, R):
    n = p0.shape[0]
    return pl.pallas_call(
        _tc2_body,
        grid=(n // R,),
        in_specs=[_row_spec(R), _row_spec(R), _deg_spec(R), _deg_spec(R),
                  _full_spec((1, F)), _full_spec((F, F))],
        out_specs=_row_spec(R),
        out_shape=jax.ShapeDtypeStruct((n, F), jnp.float32),
    )(p0, p1, din, dout, b, W)


def _tc3(q0, q1, din, b, R):
    n = q0.shape[0]
    return pl.pallas_call(
        _tc3_body,
        grid=(n // R,),
        in_specs=[_row_spec(R), _row_spec(R), _deg_spec(R),
                  _full_spec((1, F))],
        out_specs=_row_spec(R),
        out_shape=jax.ShapeDtypeStruct((n, F), jnp.float32),
    )(q0, q1, din, b)


def kernel(in_feat, edge_index, W1, b1, W2, b2):
    N = in_feat.shape[0]
    E = edge_index.shape[1]

    unit = NS * CHUNK * NB_DEG
    EP = -(-E // unit) * unit
    NPAD = 128 * (-(-(N + 1) // 128))
    pad = EP - E

    src = edge_index[0]
    dst = edge_index[1]
    if pad:
        dst_d = jnp.concatenate([dst, jnp.full((pad,), N, jnp.int32)])
        src_d = jnp.concatenate([src, jnp.full((pad,), N, jnp.int32)])
    else:
        dst_d, src_d = dst, src
    rows = EP // CHUNK
    ei_deg = jnp.stack(
        [src_d.reshape(rows, CHUNK), dst_d.reshape(rows, CHUNK)]
    )

    ES = -(-E // (4 * CHUNK)) * (4 * CHUNK)
    pad_s = ES - E
    if pad_s:
        src_g = jnp.concatenate([src, jnp.zeros((pad_s,), jnp.int32)])
        dst_p = jnp.concatenate([dst, jnp.full((pad_s,), N, jnp.int32)])
    else:
        src_g, dst_p = src, dst

    zeros_deg = jnp.zeros((NPAD,), jnp.float32)
    zeros_rows = jnp.zeros((NPAD // 8, F), jnp.float32)

    degs = _make_deg(EP, N, NPAD)(ei_deg, zeros_deg)
    dout = degs[0, :N, None]
    din = degs[1, :N, None]

    R = 1000 if N % 1000 == 0 else N
    b1r = b1[None, :]
    b2r = b2[None, :]

    spmm = _make_spmm(ES, N, NPAD)

    h1 = _tc1(in_feat, dout, W1, R)
    P = spmm(h1, src_g, dst_p, zeros_rows)
    h2 = _tc2(P[0], P[1], din, dout, b1r, W2, R)
    Q = spmm(h2, src_g, dst_p, zeros_rows)
    return _tc3(Q[0], Q[1], din, b2r, R)


# speedup vs baseline: 2.9927x; 1.0075x over previous
import functools

import jax
import jax.numpy as jnp
from jax import lax
from jax.experimental import pallas as pl
from jax.experimental.pallas import tpu as pltpu
from jax.experimental.pallas import tpu_sc as plsc

F = 128
LANES = 16
NC = 2
NS = 16
NW = NC * NS
CHUNK = 128
NB_DEG = 8
NB_SP = 3


@functools.lru_cache(maxsize=None)
def _make_deg(EP, N, NPAD):
    chunks = EP // NS // CHUNK
    groups = chunks // NB_DEG
    mesh = plsc.VectorSubcoreMesh(core_axis_name="c", subcore_axis_name="s")

    @functools.partial(
        pl.kernel,
        mesh=mesh,
        out_type=jax.ShapeDtypeStruct((2, NPAD), jnp.float32),
        scratch_types=[
            pltpu.VMEM((chunks, CHUNK), jnp.int32),
            pltpu.VMEM((CHUNK,), jnp.float32),
            pltpu.VMEM_SHARED((NPAD,), jnp.float32),
            pltpu.SemaphoreType.DMA((NB_DEG,)),
        ],
    )
    def deg(ei_hbm, zeros_hbm, out_hbm, idx_all, ones_v, acc, sems):
        c = lax.axis_index("c")
        s = lax.axis_index("s")
        for i in range(CHUNK // LANES):
            ones_v[pl.ds(i * LANES, LANES)] = jnp.full(
                (LANES,), 1.0, jnp.float32
            )

        @pl.when(s == 0)
        def _():
            pltpu.sync_copy(zeros_hbm, acc)

        pltpu.sync_copy(
            ei_hbm.at[c, pl.ds(pl.multiple_of(s * chunks, 8), chunks)],
            idx_all,
        )
        plsc.subcore_barrier()

        def body(j, carry):
            ds = []
            for b in range(NB_DEG):
                row = j * NB_DEG + b
                ds.append(pltpu.async_copy(
                    ones_v, acc.at[idx_all.at[row]], sems.at[b], add=True
                ))
            for d in ds:
                d.wait()
            return carry

        lax.fori_loop(0, groups, body, 0)
        plsc.subcore_barrier()

        @pl.when(s == 0)
        def _():
            pltpu.sync_copy(acc, out_hbm.at[c])

    return deg


@functools.lru_cache(maxsize=None)
def _make_spmm(ES, N, NPAD):
    total_pairs = ES // (2 * NB_SP * CHUNK)
    q_lo = total_pairs // NW
    q_hi = total_pairs - q_lo * NW
    zrows = NPAD // 4
    orows = 1000
    mesh = plsc.VectorSubcoreMesh(core_axis_name="c", subcore_axis_name="s")

    @functools.partial(
        pl.kernel,
        mesh=mesh,
        out_type=jax.ShapeDtypeStruct((2, N, F), jnp.float32),
        scratch_types=[
            pltpu.VMEM((2, NB_SP * CHUNK), jnp.int32),
            pltpu.VMEM((2 * NB_SP, CHUNK), jnp.int32),
            pltpu.VMEM((NB_SP, CHUNK, F), jnp.float32),
            pltpu.VMEM_SHARED((NPAD, F), jnp.float32),
            pltpu.SemaphoreType.DMA((2,)),
            pltpu.SemaphoreType.DMA((NB_SP,)),
            pltpu.SemaphoreType.DMA((NB_SP,)),
        ],
    )
    def spmm(h_hbm, src_hbm, dst_hbm, zeros_hbm, out_hbm,
             sidx, didx, rows_v, acc, isems, gsems, ssems):
        c = lax.axis_index("c")
        s = lax.axis_index("s")
        wid = c * NS + s
        npairs = q_lo + jnp.where(wid < q_hi, 1, 0)
        base = (q_lo * wid + jnp.minimum(wid, q_hi)) * (2 * NB_SP)
        groups = 2 * npairs

        def _idx_descs(par, g):
            off = (base + g * NB_SP) * CHUNK
            off = pl.multiple_of(off, 8)
            ds = [pltpu.make_async_copy(
                src_hbm.at[pl.ds(off, NB_SP * CHUNK)], sidx.at[par],
                isems.at[par],
            )]
            for b in range(NB_SP):
                offb = pl.multiple_of(off + b * CHUNK, 8)
                ds.append(pltpu.make_async_copy(
                    dst_hbm.at[pl.ds(offb, CHUNK)], didx.at[par * NB_SP + b],
                    isems.at[par],
                ))
            return ds

        def fire_idx(par, g):
            for d in _idx_descs(par, g):
                d.start()

        def drain_idx(par, g):
            for d in _idx_descs(par, g):
                d.wait()

        @pl.when(s < 4)
        def _():
            pltpu.sync_copy(
                zeros_hbm, acc.at[pl.ds(pl.multiple_of(s * zrows, 8), zrows)]
            )

        fire_idx(0, 0)
        fire_idx(1, 1)
        plsc.subcore_barrier()

        def phase(par, g):
            drain_idx(par, g)
            gs = []
            for b in range(NB_SP):
                gs.append(pltpu.async_copy(
                    h_hbm.at[sidx.at[par, pl.ds(b * CHUNK, CHUNK)]],
                    rows_v.at[b], gsems.at[b],
                ))
            ss = []
            for b in range(NB_SP):
                gs[b].wait()
                ss.append(pltpu.async_copy(
                    rows_v.at[b], acc.at[didx.at[par * NB_SP + b]],
                    ssems.at[b], add=True,
                ))
            for d in ss:
                d.wait()

            @pl.when(g + 2 < groups)
            def _():
                fire_idx(par, g + 2)

        def body(i, carry):
            phase(0, 2 * i)
            phase(1, 2 * i + 1)
            return carry

        lax.fori_loop(0, npairs, body, 0)
        plsc.subcore_barrier()

        @pl.when(s < N // orows)
        def _():
            obase = pl.multiple_of(s * orows, 8)
            pltpu.sync_copy(
                acc.at[pl.ds(obase, orows)],
                out_hbm.at[c, pl.ds(obase, orows)],
            )

    return spmm


def _tc1_body(x_ref, d_ref, w_ref, o_ref):
    s = lax.rsqrt(jnp.maximum(d_ref[...], 1.0))
    o_ref[...] = jnp.dot(
        x_ref[...] * s, w_ref[...], preferred_element_type=jnp.float32
    )


def _tc2_body(p0_ref, p1_ref, din_ref, dout_ref, b_ref, w_ref, o_ref):
    t = (p0_ref[...] + p1_ref[...]) * lax.rsqrt(
        jnp.maximum(din_ref[...], 1.0)
    ) + b_ref[...]
    t = jnp.maximum(t, 0.0)
    t = t * lax.rsqrt(jnp.maximum(dout_ref[...], 1.0))
    o_ref[...] = jnp.dot(t, w_ref[...], preferred_element_type=jnp.float32)


def _tc3_body(q0_ref, q1_ref, din_ref, b_ref, o_ref):
    o_ref[...] = (q0_ref[...] + q1_ref[...]) * lax.rsqrt(
        jnp.maximum(din_ref[...], 1.0)
    ) + b_ref[...]


def _row_spec(R):
    return pl.BlockSpec((R, F), lambda i: (i, 0))


def _deg_spec(R):
    return pl.BlockSpec((R, 1), lambda i: (i, 0))


def _full_spec(shape):
    return pl.BlockSpec(shape, lambda i: (0,) * len(shape))


def _tc1(x, dout, W, R):
    n = x.shape[0]
    return pl.pallas_call(
        _tc1_body,
        grid=(n // R,),
        in_specs=[_row_spec(R), _deg_spec(R), _full_spec((F, F))],
        out_specs=_row_spec(R),
        out_shape=jax.ShapeDtypeStruct((n, F), jnp.float32),
    )(x, dout, W)


def _tc2(p0, p1, din, dout, b, W, R):
    n = p0.shape[0]
    return pl.pallas_call(
        _tc2_body,
        grid=(n // R,),
        in_specs=[_row_spec(R), _row_spec(R), _deg_spec(R), _deg_spec(R),
                  _full_spec((1, F)), _full_spec((F, F))],
        out_specs=_row_spec(R),
        out_shape=jax.ShapeDtypeStruct((n, F), jnp.float32),
    )(p0, p1, din, dout, b, W)


def _tc3(q0, q1, din, b, R):
    n = q0.shape[0]
    return pl.pallas_call(
        _tc3_body,
        grid=(n // R,),
        in_specs=[_row_spec(R), _row_spec(R), _deg_spec(R),
                  _full_spec((1, F))],
        out_specs=_row_spec(R),
        out_shape=jax.ShapeDtypeStruct((n, F), jnp.float32),
    )(q0, q1, din, b)


def kernel(in_feat, edge_index, W1, b1, W2, b2):
    N = in_feat.shape[0]
    E = edge_index.shape[1]

    unit = NS * CHUNK * NB_DEG
    EP = -(-E // unit) * unit
    NPAD = 32 * (-(-(N + 1) // 32))
    pad = EP - E

    src = edge_index[0]
    dst = edge_index[1]
    if pad:
        dst_d = jnp.concatenate([dst, jnp.full((pad,), N, jnp.int32)])
        src_d = jnp.concatenate([src, jnp.full((pad,), N, jnp.int32)])
    else:
        dst_d, src_d = dst, src
    rows = EP // CHUNK
    ei_deg = jnp.stack(
        [src_d.reshape(rows, CHUNK), dst_d.reshape(rows, CHUNK)]
    )

    ES = -(-E // (2 * NB_SP * CHUNK)) * (2 * NB_SP * CHUNK)
    pad_s = ES - E
    if pad_s:
        src_g = jnp.concatenate([src, jnp.zeros((pad_s,), jnp.int32)])
        dst_p = jnp.concatenate([dst, jnp.full((pad_s,), N, jnp.int32)])
    else:
        src_g, dst_p = src, dst

    zeros_deg = jnp.zeros((NPAD,), jnp.float32)
    zeros_rows = jnp.zeros((NPAD // 4, F), jnp.float32)

    degs = _make_deg(EP, N, NPAD)(ei_deg, zeros_deg)
    dout = degs[0, :N, None]
    din = degs[1, :N, None]

    R = 1000 if N % 1000 == 0 else N
    b1r = b1[None, :]
    b2r = b2[None, :]

    spmm = _make_spmm(ES, N, NPAD)

    h1 = _tc1(in_feat, dout, W1, R)
    P = spmm(h1, src_g, dst_p, zeros_rows)
    h2 = _tc2(P[0], P[1], din, dout, b1r, W2, R)
    Q = spmm(h2, src_g, dst_p, zeros_rows)
    return _tc3(Q[0], Q[1], din, b2r, R)
